# baseline (speedup 1.0000x reference)
import functools

import jax
import jax.numpy as jnp
from jax import lax
from jax.experimental import pallas as pl
from jax.experimental.pallas import tpu as pltpu
from jax.experimental.pallas import tpu_sc as plsc

HIDDEN = 128
NC = 2
NS = 16
NW = NC * NS
CHUNK = 80
NPH = 5
NB = 5
NBS = 4


def _ln(h, g, b):
    n = h.shape[-1]
    ones = jnp.full((n, 1), 1.0 / n, dtype=jnp.float32)
    mu = _dot(h, ones)
    msq = _dot(h * h, ones)
    var = msq - mu * mu
    return (h - mu) * lax.rsqrt(var + 1e-5) * g + b


def _dot(a, b):
    return jnp.dot(a, b, preferred_element_type=jnp.float32)


def _gridmlp_body(x_ref, we1s_ref, wg1_ref, bg1_ref, wg2_ref, bg2_ref,
                  gg_ref, bbg_ref, p_ref, out_ref):
    x = x_ref[...]
    p_ref[...] = _dot(x, we1s_ref[...])
    h = _dot(x, wg1_ref[...]) + bg1_ref[...]
    h = h * jax.nn.sigmoid(h)
    h2 = _dot(h, wg2_ref[...]) + bg2_ref[...]
    out_ref[...] = x + _ln(h2, gg_ref[...], bbg_ref[...])


def _proj_body(m_ref, we1d_ref, p_ref):
    p_ref[...] = _dot(m_ref[...], we1d_ref[...])


def _edge_body(e_ref, s_ref, we1e_ref, be1_ref, we2_ref, be2_ref,
               ge_ref, bbe_ref, out_ref):
    e = e_ref[...]
    h = _dot(e, we1e_ref[...]) + s_ref[...] + be1_ref[...]
    h = h * jax.nn.sigmoid(h)
    h2 = _dot(h, we2_ref[...]) + be2_ref[...]
    out_ref[...] = e + _ln(h2, ge_ref[...], bbe_ref[...])


def _node_body(*refs):
    parts = refs[:2 * NPH]
    (m_ref, wn1a_ref, wn1m_ref, bn1_ref, wn2_ref, bn2_ref, gn_ref,
     bbn_ref, out_ref) = refs[2 * NPH:]
    agg = parts[0][...][0]
    for pr in parts[1:]:
        agg = agg + pr[...][0]
    m = m_ref[...]
    h = _dot(agg, wn1a_ref[...]) + _dot(m, wn1m_ref[...]) + bn1_ref[...]
    h = h * jax.nn.sigmoid(h)
    h2 = _dot(h, wn2_ref[...]) + bn2_ref[...]
    out_ref[...] = m + _ln(h2, gn_ref[...], bbn_ref[...])


def _full(shape):
    return pl.BlockSpec(shape, lambda i: (0,) * len(shape))


def _rows(br):
    return pl.BlockSpec((br, HIDDEN), lambda i: (i, 0))


def _sc_mesh():
    return plsc.VectorSubcoreMesh(core_axis_name="c", subcore_axis_name="s",
                                  num_cores=NC, num_subcores=NS)


def _make_gather(n_edge, epw_total, phase, n_chunks):
    pe = n_chunks * CHUNK
    n_out = NW * pe
    n_groups = n_chunks // NB
    assert n_chunks % NB == 0

    @functools.partial(
        pl.kernel,
        out_type=jax.ShapeDtypeStruct((n_out, HIDDEN), jnp.float32),
        mesh=_sc_mesh(),
        scratch_types=[
            pltpu.VMEM((pe,), jnp.int32),
            pltpu.VMEM((pe,), jnp.int32),
            pltpu.VMEM((NB * CHUNK, HIDDEN), jnp.float32),
            pltpu.VMEM((NB * CHUNK, HIDDEN), jnp.float32),
        ] + [pltpu.SemaphoreType.DMA] * (2 * NB),
    )
    def gather_k(psrc_hbm, pdst_hbm, srci_hbm, dsti_hbm, s_hbm,
                 idx1, idx2, rows1, rows2, *sems):
        wid = lax.axis_index("s") * NC + lax.axis_index("c")
        ibase = phase * (NW * pe) + wid * pe
        obase = wid * pe

        pltpu.sync_copy(srci_hbm.at[pl.ds(ibase, pe)], idx1)
        pltpu.sync_copy(dsti_hbm.at[pl.ds(ibase, pe)], idx2)

        def fire(c, b):
            csl = pl.ds(c * CHUNK, CHUNK)
            bsl = pl.ds(b * CHUNK, CHUNK)
            pltpu.async_copy(psrc_hbm.at[idx1.at[csl]], rows1.at[bsl], sems[b])
            pltpu.async_copy(pdst_hbm.at[idx2.at[csl]], rows2.at[bsl],
                             sems[NB + b])

        for b in range(NB):
            fire(b, b)

        def body(g, carry):
            for b in range(NB):
                c = g * NB + b
                bsl = pl.ds(b * CHUNK, CHUNK)
                hb = pl.ds(obase + c * CHUNK, CHUNK)
                csl = pl.ds(c * CHUNK, CHUNK)
                pltpu.make_async_copy(psrc_hbm.at[idx1.at[csl]],
                                      rows1.at[bsl], sems[b]).wait()
                pltpu.make_async_copy(pdst_hbm.at[idx2.at[csl]],
                                      rows2.at[bsl], sems[NB + b]).wait()

                def add_row(jj, cr):
                    row = b * CHUNK + jj
                    for k in range(HIDDEN // 16):
                        sl = pl.ds(k * 16, 16)
                        rows1[row, sl] = rows1[row, sl] + rows2[row, sl]
                    return cr

                lax.fori_loop(0, CHUNK, add_row, 0)

                pltpu.async_copy(rows1.at[bsl], s_hbm.at[hb], sems[b])
                pltpu.make_async_copy(rows1.at[bsl], s_hbm.at[hb],
                                      sems[b]).wait()

                @pl.when(g < n_groups - 1)
                def _():
                    fire(c + NB, b)
            return carry

        lax.fori_loop(0, n_groups, body, 0)

    return gather_k


def _make_scatter(n_mesh_pad, phase, n_chunks):
    rows_per_tile = n_mesh_pad // NS
    pe = n_chunks * CHUNK
    n_groups = (n_chunks + NBS - 1) // NBS

    @functools.partial(
        pl.kernel,
        out_type=jax.ShapeDtypeStruct((NC, n_mesh_pad, HIDDEN), jnp.float32),
        mesh=_sc_mesh(),
        scratch_types=[
            pltpu.VMEM((n_chunks, CHUNK), jnp.int32),
            pltpu.VMEM((NBS * CHUNK, HIDDEN), jnp.float32),
            pltpu.VMEM_SHARED((n_mesh_pad, HIDDEN), jnp.float32),
        ] + [pltpu.SemaphoreType.DMA] * NBS,
    )
    def scatter_k(e_hbm, dsti4_hbm, zeros_hbm, out_hbm, idxall, rows, acc,
                  *sems):
        cid = lax.axis_index("c")
        sid = lax.axis_index("s")
        wid = sid * NC + cid
        base0 = wid * pe
        my_rows = pl.ds(sid * rows_per_tile, rows_per_tile)

        pltpu.sync_copy(dsti4_hbm.at[phase, wid], idxall)

        def fire(c, b):
            pltpu.async_copy(e_hbm.at[pl.ds(base0 + c * CHUNK, CHUNK)],
                             rows.at[pl.ds(b * CHUNK, CHUNK)], sems[b])

        for b in range(NBS):
            fire(b, b)

        pltpu.sync_copy(zeros_hbm, acc.at[my_rows])
        plsc.subcore_barrier()

        def body(g, carry):
            for b in range(NBS):
                c = g * NBS + b
                bsl = pl.ds(b * CHUNK, CHUNK)

                @pl.when(c < n_chunks)
                def _():
                    pltpu.make_async_copy(
                        e_hbm.at[pl.ds(base0 + c * CHUNK, CHUNK)],
                        rows.at[bsl], sems[b]).wait()
                    pltpu.sync_copy(rows.at[bsl], acc.at[idxall.at[c]],
                                    add=True)

                @pl.when(c + NBS < n_chunks)
                def _():
                    fire(c + NBS, b)
            return carry

        lax.fori_loop(0, n_groups, body, 0)
        plsc.subcore_barrier()

        pltpu.sync_copy(acc.at[my_rows], out_hbm.at[cid, my_rows])

    return scatter_k


def kernel(grid_node_features, mesh_node_features, grid2mesh_edge_features,
           grid2mesh_edge_indices_src, grid2mesh_edge_indices_dst,
           We1, be1, We2, be2, ge, bbe,
           Wn1, bn1, Wn2, bn2, gn, bbn,
           Wg1, bg1, Wg2, bg2, gg, bbg):
    n_grid, hid = grid_node_features.shape
    n_mesh = mesh_node_features.shape[0]
    n_edge = grid2mesh_edge_features.shape[0]
    assert hid == HIDDEN
    assert n_edge % (NW * CHUNK * NPH) == 0 and n_mesh % NS == 0
    epw_total = n_edge // NW
    n_chunks = epw_total // (CHUNK * NPH)
    pe = n_chunks * CHUNK
    n_ph_edges = NW * pe

    We1_e, We1_s, We1_d = We1[:hid], We1[hid:2 * hid], We1[2 * hid:]
    Wn1_a, Wn1_m = Wn1[:hid], Wn1[hid:]
    r = lambda v: v.reshape(1, hid)

    br_m = n_mesh
    pdst = pl.pallas_call(
        _proj_body,
        grid=(n_mesh // br_m,),
        in_specs=[_rows(br_m), _full((hid, hid))],
        out_specs=_rows(br_m),
        out_shape=jax.ShapeDtypeStruct((n_mesh, hid), jnp.float32),
    )(mesh_node_features, We1_d)

    br_g = 4000
    psrc, grid_out = pl.pallas_call(
        _gridmlp_body,
        grid=(n_grid // br_g,),
        in_specs=[_rows(br_g), _full((hid, hid)), _full((hid, hid)),
                  _full((1, hid)), _full((hid, hid)), _full((1, hid)),
                  _full((1, hid)), _full((1, hid))],
        out_specs=[_rows(br_g), _rows(br_g)],
        out_shape=[jax.ShapeDtypeStruct((n_grid, hid), jnp.float32),
                   jax.ShapeDtypeStruct((n_grid, hid), jnp.float32)],
    )(grid_node_features, We1_s, Wg1, r(bg1), Wg2, r(bg2), r(gg), r(bbg))

    n_mesh_pad = ((n_mesh + NS * 8 - 1) // (NS * 8)) * NS * 8
    zeros_tile = jnp.zeros((n_mesh_pad // NS, hid), jnp.float32)
    dsti4 = grid2mesh_edge_indices_dst.reshape(NPH, NW, n_chunks, CHUNK)

    partials = []
    for p in range(NPH):
        sp = _make_gather(n_edge, epw_total, p, n_chunks)(
            psrc, pdst, grid2mesh_edge_indices_src,
            grid2mesh_edge_indices_dst)

        br_e = 4000
        nbl = n_ph_edges // br_e
        e_spec = pl.BlockSpec((br_e, hid), lambda i, p=p: (p * nbl + i, 0))
        e_p = pl.pallas_call(
            _edge_body,
            grid=(nbl,),
            in_specs=[e_spec, _rows(br_e), _full((hid, hid)),
                      _full((1, hid)), _full((hid, hid)), _full((1, hid)),
                      _full((1, hid)), _full((1, hid))],
            out_specs=_rows(br_e),
            out_shape=jax.ShapeDtypeStruct((n_ph_edges, hid), jnp.float32),
        )(grid2mesh_edge_features, sp, We1_e, r(be1), We2, r(be2),
          r(ge), r(bbe))

        partials.append(_make_scatter(n_mesh_pad, p, n_chunks)(
            e_p, dsti4, zeros_tile))

    br_n = 2000
    part_specs = []
    for p in range(NPH):
        for c in range(NC):
            part_specs.append(
                pl.BlockSpec((1, br_n, hid), lambda i, c=c: (c, i, 0)))
    mesh_out = pl.pallas_call(
        _node_body,
        grid=(n_mesh // br_n,),
        in_specs=part_specs + [_rows(br_n), _full((hid, hid)),
                               _full((hid, hid)), _full((1, hid)),
                               _full((hid, hid)), _full((1, hid)),
                               _full((1, hid)), _full((1, hid))],
        out_specs=_rows(br_n),
        out_shape=jax.ShapeDtypeStruct((n_mesh, hid), jnp.float32),
    )(*[partials[p] for p in range(NPH) for _ in range(NC)],
      mesh_node_features, Wn1_a, Wn1_m, r(bn1), Wn2, r(bn2), r(gn), r(bbn))

    return (grid_out, mesh_out)


# speedup vs baseline: 1.0057x; 1.0057x over previous
import functools

import jax
import jax.numpy as jnp
from jax import lax
from jax.experimental import pallas as pl
from jax.experimental.pallas import tpu as pltpu
from jax.experimental.pallas import tpu_sc as plsc

HIDDEN = 128
NC = 2
NS = 16
NW = NC * NS
CHUNK = 80
NPH = 5
NB = 5
NBS = 4


def _ln(h, g, b):
    mu = jnp.mean(h, axis=-1, keepdims=True)
    d = h - mu
    var = jnp.mean(d * d, axis=-1, keepdims=True)
    return d * lax.rsqrt(var + 1e-5) * g + b


def _dot(a, b):
    return jnp.dot(a, b, preferred_element_type=jnp.float32)


def _gridmlp_body(x_ref, we1s_ref, wg1_ref, bg1_ref, wg2_ref, bg2_ref,
                  gg_ref, bbg_ref, p_ref, out_ref):
    x = x_ref[...]
    p_ref[...] = _dot(x, we1s_ref[...])
    h = _dot(x, wg1_ref[...]) + bg1_ref[...]
    h = h * jax.nn.sigmoid(h)
    h2 = _dot(h, wg2_ref[...]) + bg2_ref[...]
    out_ref[...] = x + _ln(h2, gg_ref[...], bbg_ref[...])


def _proj_body(m_ref, we1d_ref, p_ref):
    p_ref[...] = _dot(m_ref[...], we1d_ref[...])


def _edge_body(e_ref, s_ref, we1e_ref, be1_ref, we2_ref, be2_ref,
               ge_ref, bbe_ref, out_ref):
    e = e_ref[...]
    h = _dot(e, we1e_ref[...]) + s_ref[...] + be1_ref[...]
    h = h * jax.nn.sigmoid(h)
    h2 = _dot(h, we2_ref[...]) + be2_ref[...]
    out_ref[...] = e + _ln(h2, ge_ref[...], bbe_ref[...])


def _node_body(*refs):
    parts = refs[:2 * NPH]
    (m_ref, wn1a_ref, wn1m_ref, bn1_ref, wn2_ref, bn2_ref, gn_ref,
     bbn_ref, out_ref) = refs[2 * NPH:]
    agg = parts[0][...][0]
    for pr in parts[1:]:
        agg = agg + pr[...][0]
    m = m_ref[...]
    h = _dot(agg, wn1a_ref[...]) + _dot(m, wn1m_ref[...]) + bn1_ref[...]
    h = h * jax.nn.sigmoid(h)
    h2 = _dot(h, wn2_ref[...]) + bn2_ref[...]
    out_ref[...] = m + _ln(h2, gn_ref[...], bbn_ref[...])


def _full(shape):
    return pl.BlockSpec(shape, lambda i: (0,) * len(shape))


def _rows(br):
    return pl.BlockSpec((br, HIDDEN), lambda i: (i, 0))


def _sc_mesh():
    return plsc.VectorSubcoreMesh(core_axis_name="c", subcore_axis_name="s",
                                  num_cores=NC, num_subcores=NS)


def _make_gather(n_edge, epw_total, phase, n_chunks):
    pe = n_chunks * CHUNK
    n_out = NW * pe
    n_groups = n_chunks // NB
    assert n_chunks % NB == 0

    @functools.partial(
        pl.kernel,
        out_type=jax.ShapeDtypeStruct((n_out, HIDDEN), jnp.float32),
        mesh=_sc_mesh(),
        scratch_types=[
            pltpu.VMEM((pe,), jnp.int32),
            pltpu.VMEM((pe,), jnp.int32),
            pltpu.VMEM((NB * CHUNK, HIDDEN), jnp.float32),
            pltpu.VMEM((NB * CHUNK, HIDDEN), jnp.float32),
        ] + [pltpu.SemaphoreType.DMA] * (2 * NB),
    )
    def gather_k(psrc_hbm, pdst_hbm, srci_hbm, dsti_hbm, s_hbm,
                 idx1, idx2, rows1, rows2, *sems):
        wid = lax.axis_index("s") * NC + lax.axis_index("c")
        ibase = phase * (NW * pe) + wid * pe
        obase = wid * pe

        pltpu.sync_copy(srci_hbm.at[pl.ds(ibase, pe)], idx1)
        pltpu.sync_copy(dsti_hbm.at[pl.ds(ibase, pe)], idx2)

        def fire(c, b):
            csl = pl.ds(c * CHUNK, CHUNK)
            bsl = pl.ds(b * CHUNK, CHUNK)
            pltpu.async_copy(psrc_hbm.at[idx1.at[csl]], rows1.at[bsl], sems[b])
            pltpu.async_copy(pdst_hbm.at[idx2.at[csl]], rows2.at[bsl],
                             sems[NB + b])

        for b in range(NB):
            fire(b, b)

        def body(g, carry):
            for b in range(NB):
                c = g * NB + b
                bsl = pl.ds(b * CHUNK, CHUNK)
                hb = pl.ds(obase + c * CHUNK, CHUNK)
                csl = pl.ds(c * CHUNK, CHUNK)
                pltpu.make_async_copy(psrc_hbm.at[idx1.at[csl]],
                                      rows1.at[bsl], sems[b]).wait()
                pltpu.make_async_copy(pdst_hbm.at[idx2.at[csl]],
                                      rows2.at[bsl], sems[NB + b]).wait()

                def add_row(jj, cr):
                    row = b * CHUNK + jj
                    for k in range(HIDDEN // 16):
                        sl = pl.ds(k * 16, 16)
                        rows1[row, sl] = rows1[row, sl] + rows2[row, sl]
                    return cr

                lax.fori_loop(0, CHUNK, add_row, 0)

                pltpu.async_copy(rows1.at[bsl], s_hbm.at[hb], sems[b])
                pltpu.make_async_copy(rows1.at[bsl], s_hbm.at[hb],
                                      sems[b]).wait()

                @pl.when(g < n_groups - 1)
                def _():
                    fire(c + NB, b)
            return carry

        lax.fori_loop(0, n_groups, body, 0)

    return gather_k


def _make_scatter(n_mesh_pad, phase, n_chunks):
    rows_per_tile = n_mesh_pad // NS
    pe = n_chunks * CHUNK
    n_groups = (n_chunks + NBS - 1) // NBS

    @functools.partial(
        pl.kernel,
        out_type=jax.ShapeDtypeStruct((NC, n_mesh_pad, HIDDEN), jnp.float32),
        mesh=_sc_mesh(),
        scratch_types=[
            pltpu.VMEM((n_chunks, CHUNK), jnp.int32),
            pltpu.VMEM((NBS * CHUNK, HIDDEN), jnp.float32),
            pltpu.VMEM_SHARED((n_mesh_pad, HIDDEN), jnp.float32),
        ] + [pltpu.SemaphoreType.DMA] * NBS,
    )
    def scatter_k(e_hbm, dsti4_hbm, zeros_hbm, out_hbm, idxall, rows, acc,
                  *sems):
        cid = lax.axis_index("c")
        sid = lax.axis_index("s")
        wid = sid * NC + cid
        base0 = wid * pe
        my_rows = pl.ds(sid * rows_per_tile, rows_per_tile)

        pltpu.sync_copy(dsti4_hbm.at[phase, wid], idxall)

        def fire(c, b):
            pltpu.async_copy(e_hbm.at[pl.ds(base0 + c * CHUNK, CHUNK)],
                             rows.at[pl.ds(b * CHUNK, CHUNK)], sems[b])

        for b in range(NBS):
            fire(b, b)

        pltpu.sync_copy(zeros_hbm, acc.at[my_rows])
        plsc.subcore_barrier()

        def body(g, carry):
            for b in range(NBS):
                c = g * NBS + b
                bsl = pl.ds(b * CHUNK, CHUNK)

                @pl.when(c < n_chunks)
                def _():
                    pltpu.make_async_copy(
                        e_hbm.at[pl.ds(base0 + c * CHUNK, CHUNK)],
                        rows.at[bsl], sems[b]).wait()
                    pltpu.sync_copy(rows.at[bsl], acc.at[idxall.at[c]],
                                    add=True)

                @pl.when(c + NBS < n_chunks)
                def _():
                    fire(c + NBS, b)
            return carry

        lax.fori_loop(0, n_groups, body, 0)
        plsc.subcore_barrier()

        pltpu.sync_copy(acc.at[my_rows], out_hbm.at[cid, my_rows])

    return scatter_k


def kernel(grid_node_features, mesh_node_features, grid2mesh_edge_features,
           grid2mesh_edge_indices_src, grid2mesh_edge_indices_dst,
           We1, be1, We2, be2, ge, bbe,
           Wn1, bn1, Wn2, bn2, gn, bbn,
           Wg1, bg1, Wg2, bg2, gg, bbg):
    n_grid, hid = grid_node_features.shape
    n_mesh = mesh_node_features.shape[0]
    n_edge = grid2mesh_edge_features.shape[0]
    assert hid == HIDDEN
    assert n_edge % (NW * CHUNK * NPH) == 0 and n_mesh % NS == 0
    epw_total = n_edge // NW
    n_chunks = epw_total // (CHUNK * NPH)
    pe = n_chunks * CHUNK
    n_ph_edges = NW * pe

    We1_e, We1_s, We1_d = We1[:hid], We1[hid:2 * hid], We1[2 * hid:]
    Wn1_a, Wn1_m = Wn1[:hid], Wn1[hid:]
    r = lambda v: v.reshape(1, hid)

    br_m = n_mesh
    pdst = pl.pallas_call(
        _proj_body,
        grid=(n_mesh // br_m,),
        in_specs=[_rows(br_m), _full((hid, hid))],
        out_specs=_rows(br_m),
        out_shape=jax.ShapeDtypeStruct((n_mesh, hid), jnp.float32),
    )(mesh_node_features, We1_d)

    br_g = 4000
    psrc, grid_out = pl.pallas_call(
        _gridmlp_body,
        grid=(n_grid // br_g,),
        in_specs=[_rows(br_g), _full((hid, hid)), _full((hid, hid)),
                  _full((1, hid)), _full((hid, hid)), _full((1, hid)),
                  _full((1, hid)), _full((1, hid))],
        out_specs=[_rows(br_g), _rows(br_g)],
        out_shape=[jax.ShapeDtypeStruct((n_grid, hid), jnp.float32),
                   jax.ShapeDtypeStruct((n_grid, hid), jnp.float32)],
    )(grid_node_features, We1_s, Wg1, r(bg1), Wg2, r(bg2), r(gg), r(bbg))

    n_mesh_pad = ((n_mesh + NS * 8 - 1) // (NS * 8)) * NS * 8
    zeros_tile = jnp.zeros((n_mesh_pad // NS, hid), jnp.float32)
    dsti4 = grid2mesh_edge_indices_dst.reshape(NPH, NW, n_chunks, CHUNK)

    partials = []
    for p in range(NPH):
        sp = _make_gather(n_edge, epw_total, p, n_chunks)(
            psrc, pdst, grid2mesh_edge_indices_src,
            grid2mesh_edge_indices_dst)

        br_e = 4000
        nbl = n_ph_edges // br_e
        e_spec = pl.BlockSpec((br_e, hid), lambda i, p=p: (p * nbl + i, 0))
        e_p = pl.pallas_call(
            _edge_body,
            grid=(nbl,),
            in_specs=[e_spec, _rows(br_e), _full((hid, hid)),
                      _full((1, hid)), _full((hid, hid)), _full((1, hid)),
                      _full((1, hid)), _full((1, hid))],
            out_specs=_rows(br_e),
            out_shape=jax.ShapeDtypeStruct((n_ph_edges, hid), jnp.float32),
        )(grid2mesh_edge_features, sp, We1_e, r(be1), We2, r(be2),
          r(ge), r(bbe))

        partials.append(_make_scatter(n_mesh_pad, p, n_chunks)(
            e_p, dsti4, zeros_tile))

    br_n = 2000
    part_specs = []
    for p in range(NPH):
        for c in range(NC):
            part_specs.append(
                pl.BlockSpec((1, br_n, hid), lambda i, c=c: (c, i, 0)))
    mesh_out = pl.pallas_call(
        _node_body,
        grid=(n_mesh // br_n,),
        in_specs=part_specs + [_rows(br_n), _full((hid, hid)),
                               _full((hid, hid)), _full((1, hid)),
                               _full((hid, hid)), _full((1, hid)),
                               _full((1, hid)), _full((1, hid))],
        out_specs=_rows(br_n),
        out_shape=jax.ShapeDtypeStruct((n_mesh, hid), jnp.float32),
    )(*[partials[p] for p in range(NPH) for _ in range(NC)],
      mesh_node_features, Wn1_a, Wn1_m, r(bn1), Wn2, r(bn2), r(gn), r(bbn))

    return (grid_out, mesh_out)


# speedup vs baseline: 1.0222x; 1.0164x over previous
import functools

import jax
import jax.numpy as jnp
from jax import lax
from jax.experimental import pallas as pl
from jax.experimental.pallas import tpu as pltpu
from jax.experimental.pallas import tpu_sc as plsc

HIDDEN = 128
NC = 2
NS = 16
NW = NC * NS
CHUNK = 80
NPH = 5
NB = 5
NBS = 4


def _ln(h, g, b):
    mu = jnp.mean(h, axis=-1, keepdims=True)
    d = h - mu
    var = jnp.mean(d * d, axis=-1, keepdims=True)
    return d * lax.rsqrt(var + 1e-5) * g + b


def _dot(a, b):
    return jnp.dot(a, b, preferred_element_type=jnp.float32)


def _gridmlp_body(x_ref, wg1_ref, bg1_ref, wg2_ref, bg2_ref,
                  gg_ref, bbg_ref, out_ref):
    x = x_ref[...]
    h = _dot(x, wg1_ref[...]) + bg1_ref[...]
    h = h * jax.nn.sigmoid(h)
    h2 = _dot(h, wg2_ref[...]) + bg2_ref[...]
    out_ref[...] = x + _ln(h2, gg_ref[...], bbg_ref[...])


def _proj_body(m_ref, we1d_ref, p_ref):
    p_ref[...] = _dot(m_ref[...], we1d_ref[...])


def _edge_body(e_ref, s_ref, we1e_ref, be1_ref, we2_ref, be2_ref,
               ge_ref, bbe_ref, out_ref):
    e = e_ref[...]
    h = _dot(e, we1e_ref[...]) + s_ref[...] + be1_ref[...]
    h = h * jax.nn.sigmoid(h)
    h2 = _dot(h, we2_ref[...]) + be2_ref[...]
    out_ref[...] = e + _ln(h2, ge_ref[...], bbe_ref[...])


def _node_body(*refs):
    parts = refs[:2 * NPH]
    (m_ref, wn1a_ref, wn1m_ref, bn1_ref, wn2_ref, bn2_ref, gn_ref,
     bbn_ref, out_ref) = refs[2 * NPH:]
    agg = parts[0][...][0]
    for pr in parts[1:]:
        agg = agg + pr[...][0]
    m = m_ref[...]
    h = _dot(agg, wn1a_ref[...]) + _dot(m, wn1m_ref[...]) + bn1_ref[...]
    h = h * jax.nn.sigmoid(h)
    h2 = _dot(h, wn2_ref[...]) + bn2_ref[...]
    out_ref[...] = m + _ln(h2, gn_ref[...], bbn_ref[...])


def _full(shape):
    return pl.BlockSpec(shape, lambda i: (0,) * len(shape))


def _rows(br):
    return pl.BlockSpec((br, HIDDEN), lambda i: (i, 0))


def _sc_mesh():
    return plsc.VectorSubcoreMesh(core_axis_name="c", subcore_axis_name="s",
                                  num_cores=NC, num_subcores=NS)


def _make_gather(n_edge, epw_total, phase, n_chunks):
    pe = n_chunks * CHUNK
    n_out = NW * pe
    n_groups = n_chunks // NB
    assert n_chunks % NB == 0

    @functools.partial(
        pl.kernel,
        out_type=jax.ShapeDtypeStruct((n_out, HIDDEN), jnp.float32),
        mesh=_sc_mesh(),
        scratch_types=[
            pltpu.VMEM((pe,), jnp.int32),
            pltpu.VMEM((pe,), jnp.int32),
            pltpu.VMEM((NB * CHUNK, HIDDEN), jnp.float32),
            pltpu.VMEM((NB * CHUNK, HIDDEN), jnp.float32),
        ] + [pltpu.SemaphoreType.DMA] * (2 * NB),
    )
    def gather_k(psrc_hbm, pdst_hbm, srci_hbm, dsti_hbm, s_hbm,
                 idx1, idx2, rows1, rows2, *sems):
        wid = lax.axis_index("s") * NC + lax.axis_index("c")
        ibase = wid * epw_total + phase * pe
        obase = wid * pe

        pltpu.sync_copy(srci_hbm.at[pl.ds(ibase, pe)], idx1)
        pltpu.sync_copy(dsti_hbm.at[pl.ds(ibase, pe)], idx2)

        def fire(c, b):
            csl = pl.ds(c * CHUNK, CHUNK)
            bsl = pl.ds(b * CHUNK, CHUNK)
            pltpu.async_copy(psrc_hbm.at[idx1.at[csl]], rows1.at[bsl], sems[b])
            pltpu.async_copy(pdst_hbm.at[idx2.at[csl]], rows2.at[bsl],
                             sems[NB + b])

        for b in range(NB):
            fire(b, b)

        def body(g, carry):
            for b in range(NB):
                c = g * NB + b
                bsl = pl.ds(b * CHUNK, CHUNK)
                hb = pl.ds(obase + c * CHUNK, CHUNK)
                csl = pl.ds(c * CHUNK, CHUNK)
                pltpu.make_async_copy(psrc_hbm.at[idx1.at[csl]],
                                      rows1.at[bsl], sems[b]).wait()
                pltpu.make_async_copy(pdst_hbm.at[idx2.at[csl]],
                                      rows2.at[bsl], sems[NB + b]).wait()

                def add_row(jj, cr):
                    row = b * CHUNK + jj
                    for k in range(HIDDEN // 16):
                        sl = pl.ds(k * 16, 16)
                        rows1[row, sl] = rows1[row, sl] + rows2[row, sl]
                    return cr

                lax.fori_loop(0, CHUNK, add_row, 0)

                pltpu.async_copy(rows1.at[bsl], s_hbm.at[hb], sems[b])
                pltpu.make_async_copy(rows1.at[bsl], s_hbm.at[hb],
                                      sems[b]).wait()

                @pl.when(g < n_groups - 1)
                def _():
                    fire(c + NB, b)
            return carry

        lax.fori_loop(0, n_groups, body, 0)

    return gather_k


def _make_scatter(n_mesh_pad, phase, n_chunks):
    rows_per_tile = n_mesh_pad // NS
    pe = n_chunks * CHUNK
    n_groups = (n_chunks + NBS - 1) // NBS

    @functools.partial(
        pl.kernel,
        out_type=jax.ShapeDtypeStruct((NC, n_mesh_pad, HIDDEN), jnp.float32),
        mesh=_sc_mesh(),
        scratch_types=[
            pltpu.VMEM((n_chunks, CHUNK), jnp.int32),
            pltpu.VMEM((NBS * CHUNK, HIDDEN), jnp.float32),
            pltpu.VMEM_SHARED((n_mesh_pad, HIDDEN), jnp.float32),
        ] + [pltpu.SemaphoreType.DMA] * NBS,
    )
    def scatter_k(e_hbm, dsti4_hbm, zeros_hbm, out_hbm, idxall, rows, acc,
                  *sems):
        cid = lax.axis_index("c")
        sid = lax.axis_index("s")
        wid = sid * NC + cid
        base0 = wid * pe
        my_rows = pl.ds(sid * rows_per_tile, rows_per_tile)

        pltpu.sync_copy(dsti4_hbm.at[wid, phase], idxall)

        def fire(c, b):
            pltpu.async_copy(e_hbm.at[pl.ds(base0 + c * CHUNK, CHUNK)],
                             rows.at[pl.ds(b * CHUNK, CHUNK)], sems[b])

        for b in range(NBS):
            fire(b, b)

        pltpu.sync_copy(zeros_hbm, acc.at[my_rows])
        plsc.subcore_barrier()

        def body(g, carry):
            for b in range(NBS):
                c = g * NBS + b
                bsl = pl.ds(b * CHUNK, CHUNK)

                @pl.when(c < n_chunks)
                def _():
                    pltpu.make_async_copy(
                        e_hbm.at[pl.ds(base0 + c * CHUNK, CHUNK)],
                        rows.at[bsl], sems[b]).wait()
                    pltpu.sync_copy(rows.at[bsl], acc.at[idxall.at[c]],
                                    add=True)

                @pl.when(c + NBS < n_chunks)
                def _():
                    fire(c + NBS, b)
            return carry

        lax.fori_loop(0, n_groups, body, 0)
        plsc.subcore_barrier()

        pltpu.sync_copy(acc.at[my_rows], out_hbm.at[cid, my_rows])

    return scatter_k


def kernel(grid_node_features, mesh_node_features, grid2mesh_edge_features,
           grid2mesh_edge_indices_src, grid2mesh_edge_indices_dst,
           We1, be1, We2, be2, ge, bbe,
           Wn1, bn1, Wn2, bn2, gn, bbn,
           Wg1, bg1, Wg2, bg2, gg, bbg):
    n_grid, hid = grid_node_features.shape
    n_mesh = mesh_node_features.shape[0]
    n_edge = grid2mesh_edge_features.shape[0]
    assert hid == HIDDEN
    assert n_edge % (NW * CHUNK * NPH) == 0 and n_mesh % NS == 0
    epw_total = n_edge // NW
    n_chunks = epw_total // (CHUNK * NPH)
    pe = n_chunks * CHUNK
    n_ph_edges = NW * pe

    We1_e, We1_s, We1_d = We1[:hid], We1[hid:2 * hid], We1[2 * hid:]
    Wn1_a, Wn1_m = Wn1[:hid], Wn1[hid:]
    r = lambda v: v.reshape(1, hid)

    br_g = 2000
    psrc = pl.pallas_call(
        _proj_body,
        grid=(n_grid // br_g,),
        in_specs=[_rows(br_g), _full((hid, hid))],
        out_specs=_rows(br_g),
        out_shape=jax.ShapeDtypeStruct((n_grid, hid), jnp.float32),
    )(grid_node_features, We1_s)

    br_m = 2000
    pdst = pl.pallas_call(
        _proj_body,
        grid=(n_mesh // br_m,),
        in_specs=[_rows(br_m), _full((hid, hid))],
        out_specs=_rows(br_m),
        out_shape=jax.ShapeDtypeStruct((n_mesh, hid), jnp.float32),
    )(mesh_node_features, We1_d)

    grid_out = pl.pallas_call(
        _gridmlp_body,
        grid=(n_grid // br_g,),
        in_specs=[_rows(br_g), _full((hid, hid)), _full((1, hid)),
                  _full((hid, hid)), _full((1, hid)), _full((1, hid)),
                  _full((1, hid))],
        out_specs=_rows(br_g),
        out_shape=jax.ShapeDtypeStruct((n_grid, hid), jnp.float32),
    )(grid_node_features, Wg1, r(bg1), Wg2, r(bg2), r(gg), r(bbg))

    n_mesh_pad = ((n_mesh + NS * 8 - 1) // (NS * 8)) * NS * 8
    zeros_tile = jnp.zeros((n_mesh_pad // NS, hid), jnp.float32)
    dsti4 = grid2mesh_edge_indices_dst.reshape(NW, NPH, n_chunks, CHUNK)

    partials = []
    for p in range(NPH):
        sp = _make_gather(n_edge, epw_total, p, n_chunks)(
            psrc, pdst, grid2mesh_edge_indices_src,
            grid2mesh_edge_indices_dst)

        e_spec = pl.BlockSpec((pe, hid), lambda i, p=p: (i * NPH + p, 0))
        e_p = pl.pallas_call(
            _edge_body,
            grid=(NW,),
            in_specs=[e_spec, _rows(pe), _full((hid, hid)),
                      _full((1, hid)), _full((hid, hid)), _full((1, hid)),
                      _full((1, hid)), _full((1, hid))],
            out_specs=_rows(pe),
            out_shape=jax.ShapeDtypeStruct((n_ph_edges, hid), jnp.float32),
        )(grid2mesh_edge_features, sp, We1_e, r(be1), We2, r(be2),
          r(ge), r(bbe))

        partials.append(_make_scatter(n_mesh_pad, p, n_chunks)(
            e_p, dsti4, zeros_tile))

    br_n = 2000
    part_specs = []
    for p in range(NPH):
        for c in range(NC):
            part_specs.append(
                pl.BlockSpec((1, br_n, hid), lambda i, c=c: (c, i, 0)))
    mesh_out = pl.pallas_call(
        _node_body,
        grid=(n_mesh // br_n,),
        in_specs=part_specs + [_rows(br_n), _full((hid, hid)),
                               _full((hid, hid)), _full((1, hid)),
                               _full((hid, hid)), _full((1, hid)),
                               _full((1, hid)), _full((1, hid))],
        out_specs=_rows(br_n),
        out_shape=jax.ShapeDtypeStruct((n_mesh, hid), jnp.float32),
    )(*[partials[p] for p in range(NPH) for _ in range(NC)],
      mesh_node_features, Wn1_a, Wn1_m, r(bn1), Wn2, r(bn2), r(gn), r(bbn))

    return (grid_out, mesh_out)


# speedup vs baseline: 1.0852x; 1.0617x over previous
import functools

import jax
import jax.numpy as jnp
from jax import lax
from jax.experimental import pallas as pl
from jax.experimental.pallas import tpu as pltpu
from jax.experimental.pallas import tpu_sc as plsc

HIDDEN = 128
NC = 2
NS = 16
NW = NC * NS
CHUNK = 80
NPH = 5
NB = 5
NBS = 3


def _ln(h, g, b):
    mu = jnp.mean(h, axis=-1, keepdims=True)
    d = h - mu
    var = jnp.mean(d * d, axis=-1, keepdims=True)
    return d * lax.rsqrt(var + 1e-5) * g + b


def _dot(a, b):
    return jnp.dot(a, b, preferred_element_type=jnp.float32)


def _gridmlp_body(x_ref, wg1_ref, bg1_ref, wg2_ref, bg2_ref,
                  gg_ref, bbg_ref, out_ref):
    x = x_ref[...]
    h = _dot(x, wg1_ref[...]) + bg1_ref[...]
    h = h * jax.nn.sigmoid(h)
    h2 = _dot(h, wg2_ref[...]) + bg2_ref[...]
    out_ref[...] = x + _ln(h2, gg_ref[...], bbg_ref[...])


def _proj_body(m_ref, we1d_ref, p_ref):
    p_ref[...] = _dot(m_ref[...], we1d_ref[...])


def _edge_body(e_ref, s_ref, we1e_ref, be1_ref, we2_ref, be2_ref,
               ge_ref, bbe_ref, out_ref):
    e = e_ref[...]
    h = _dot(e, we1e_ref[...]) + s_ref[...] + be1_ref[...]
    h = h * jax.nn.sigmoid(h)
    h2 = _dot(h, we2_ref[...]) + be2_ref[...]
    out_ref[...] = e + _ln(h2, ge_ref[...], bbe_ref[...])


def _node_body(*refs):
    parts = refs[:-9]
    (m_ref, wn1a_ref, wn1m_ref, bn1_ref, wn2_ref, bn2_ref, gn_ref,
     bbn_ref, out_ref) = refs[-9:]
    agg = parts[0][...][0]
    for pr in parts[1:]:
        agg = agg + pr[...][0]
    m = m_ref[...]
    h = _dot(agg, wn1a_ref[...]) + _dot(m, wn1m_ref[...]) + bn1_ref[...]
    h = h * jax.nn.sigmoid(h)
    h2 = _dot(h, wn2_ref[...]) + bn2_ref[...]
    out_ref[...] = m + _ln(h2, gn_ref[...], bbn_ref[...])


def _full(shape):
    return pl.BlockSpec(shape, lambda i: (0,) * len(shape))


def _rows(br):
    return pl.BlockSpec((br, HIDDEN), lambda i: (i, 0))


def _sc_mesh():
    return plsc.VectorSubcoreMesh(core_axis_name="c", subcore_axis_name="s",
                                  num_cores=NC, num_subcores=NS)


def _make_gather(n_edge, epw_total, phase, n_chunks):
    pe = n_chunks * CHUNK
    n_out = NW * pe
    n_groups = n_chunks // NB
    assert n_chunks % NB == 0

    @functools.partial(
        pl.kernel,
        out_type=jax.ShapeDtypeStruct((n_out, HIDDEN), jnp.float32),
        mesh=_sc_mesh(),
        scratch_types=[
            pltpu.VMEM((pe,), jnp.int32),
            pltpu.VMEM((pe,), jnp.int32),
            pltpu.VMEM((NB * CHUNK, HIDDEN), jnp.float32),
            pltpu.VMEM((NB * CHUNK, HIDDEN), jnp.float32),
        ] + [pltpu.SemaphoreType.DMA] * (2 * NB),
    )
    def gather_k(psrc_hbm, pdst_hbm, srci_hbm, dsti_hbm, s_hbm,
                 idx1, idx2, rows1, rows2, *sems):
        wid = lax.axis_index("s") * NC + lax.axis_index("c")
        ibase = wid * epw_total + phase * pe
        obase = wid * pe

        pltpu.sync_copy(srci_hbm.at[pl.ds(ibase, pe)], idx1)
        pltpu.sync_copy(dsti_hbm.at[pl.ds(ibase, pe)], idx2)

        def fire(c, b):
            csl = pl.ds(c * CHUNK, CHUNK)
            bsl = pl.ds(b * CHUNK, CHUNK)
            pltpu.async_copy(psrc_hbm.at[idx1.at[csl]], rows1.at[bsl], sems[b])
            pltpu.async_copy(pdst_hbm.at[idx2.at[csl]], rows2.at[bsl],
                             sems[NB + b])

        for b in range(NB):
            fire(b, b)

        def body(g, carry):
            for b in range(NB):
                c = g * NB + b
                bsl = pl.ds(b * CHUNK, CHUNK)
                hb = pl.ds(obase + c * CHUNK, CHUNK)
                csl = pl.ds(c * CHUNK, CHUNK)
                pltpu.make_async_copy(psrc_hbm.at[idx1.at[csl]],
                                      rows1.at[bsl], sems[b]).wait()
                pltpu.make_async_copy(pdst_hbm.at[idx2.at[csl]],
                                      rows2.at[bsl], sems[NB + b]).wait()

                def add_row(jj, cr):
                    row = b * CHUNK + jj
                    for k in range(HIDDEN // 16):
                        sl = pl.ds(k * 16, 16)
                        rows1[row, sl] = rows1[row, sl] + rows2[row, sl]
                    return cr

                lax.fori_loop(0, CHUNK, add_row, 0)

                pltpu.async_copy(rows1.at[bsl], s_hbm.at[hb], sems[b])
                pltpu.make_async_copy(rows1.at[bsl], s_hbm.at[hb],
                                      sems[b]).wait()

                @pl.when(g < n_groups - 1)
                def _():
                    fire(c + NB, b)
            return carry

        lax.fori_loop(0, n_groups, body, 0)

    return gather_k


def _make_scatter(n_mesh_pad, phases, n_chunks):
    rows_per_tile = n_mesh_pad // NS
    pe = n_chunks * CHUNK
    npz = len(phases)
    n_groups = (n_chunks + NBS - 1) // NBS

    @functools.partial(
        pl.kernel,
        out_type=jax.ShapeDtypeStruct((NC, n_mesh_pad, HIDDEN), jnp.float32),
        mesh=_sc_mesh(),
        scratch_types=[
            pltpu.VMEM((npz * n_chunks, CHUNK), jnp.int32),
            pltpu.VMEM((NBS * CHUNK, HIDDEN), jnp.float32),
            pltpu.VMEM_SHARED((n_mesh_pad, HIDDEN), jnp.float32),
        ] + [pltpu.SemaphoreType.DMA] * NBS,
    )
    def scatter_k(*args):
        e_hbms = args[:npz]
        dsti4_hbm, zeros_hbm, out_hbm, idxall, rows, acc = args[npz:npz + 6]
        sems = args[npz + 6:]
        cid = lax.axis_index("c")
        sid = lax.axis_index("s")
        wid = sid * NC + cid
        base0 = wid * pe
        my_rows = pl.ds(sid * rows_per_tile, rows_per_tile)

        for j, ph in enumerate(phases):
            pltpu.sync_copy(dsti4_hbm.at[wid, ph],
                            idxall.at[pl.ds(j * n_chunks, n_chunks)])

        def fire(e_hbm, c, b):
            pltpu.async_copy(e_hbm.at[pl.ds(base0 + c * CHUNK, CHUNK)],
                             rows.at[pl.ds(b * CHUNK, CHUNK)], sems[b])

        for b in range(NBS):
            fire(e_hbms[0], b, b)

        pltpu.sync_copy(zeros_hbm, acc.at[my_rows])
        plsc.subcore_barrier()

        for j in range(npz):
            e_hbm = e_hbms[j]
            if j > 0:
                for b in range(NBS):
                    fire(e_hbm, b, b)

            def body(g, carry, e_hbm=e_hbm, j=j):
                for b in range(NBS):
                    c = g * NBS + b
                    bsl = pl.ds(b * CHUNK, CHUNK)

                    @pl.when(c < n_chunks)
                    def _():
                        pltpu.make_async_copy(
                            e_hbm.at[pl.ds(base0 + c * CHUNK, CHUNK)],
                            rows.at[bsl], sems[b]).wait()
                        pltpu.sync_copy(
                            rows.at[bsl],
                            acc.at[idxall.at[j * n_chunks + c]], add=True)

                    @pl.when(c + NBS < n_chunks)
                    def _():
                        fire(e_hbm, c + NBS, b)
                return carry

            lax.fori_loop(0, n_groups, body, 0)

        plsc.subcore_barrier()

        pltpu.sync_copy(acc.at[my_rows], out_hbm.at[cid, my_rows])

    return scatter_k


def kernel(grid_node_features, mesh_node_features, grid2mesh_edge_features,
           grid2mesh_edge_indices_src, grid2mesh_edge_indices_dst,
           We1, be1, We2, be2, ge, bbe,
           Wn1, bn1, Wn2, bn2, gn, bbn,
           Wg1, bg1, Wg2, bg2, gg, bbg):
    n_grid, hid = grid_node_features.shape
    n_mesh = mesh_node_features.shape[0]
    n_edge = grid2mesh_edge_features.shape[0]
    assert hid == HIDDEN
    assert n_edge % (NW * CHUNK * NPH) == 0 and n_mesh % NS == 0
    epw_total = n_edge // NW
    n_chunks = epw_total // (CHUNK * NPH)
    pe = n_chunks * CHUNK
    n_ph_edges = NW * pe

    We1_e, We1_s, We1_d = We1[:hid], We1[hid:2 * hid], We1[2 * hid:]
    Wn1_a, Wn1_m = Wn1[:hid], Wn1[hid:]
    r = lambda v: v.reshape(1, hid)

    br_g = 2000
    psrc = pl.pallas_call(
        _proj_body,
        grid=(n_grid // br_g,),
        in_specs=[_rows(br_g), _full((hid, hid))],
        out_specs=_rows(br_g),
        out_shape=jax.ShapeDtypeStruct((n_grid, hid), jnp.float32),
    )(grid_node_features, We1_s)

    br_m = 2000
    pdst = pl.pallas_call(
        _proj_body,
        grid=(n_mesh // br_m,),
        in_specs=[_rows(br_m), _full((hid, hid))],
        out_specs=_rows(br_m),
        out_shape=jax.ShapeDtypeStruct((n_mesh, hid), jnp.float32),
    )(mesh_node_features, We1_d)

    grid_out = pl.pallas_call(
        _gridmlp_body,
        grid=(n_grid // br_g,),
        in_specs=[_rows(br_g), _full((hid, hid)), _full((1, hid)),
                  _full((hid, hid)), _full((1, hid)), _full((1, hid)),
                  _full((1, hid))],
        out_specs=_rows(br_g),
        out_shape=jax.ShapeDtypeStruct((n_grid, hid), jnp.float32),
    )(grid_node_features, Wg1, r(bg1), Wg2, r(bg2), r(gg), r(bbg))

    n_mesh_pad = ((n_mesh + NS * 8 - 1) // (NS * 8)) * NS * 8
    zeros_tile = jnp.zeros((n_mesh_pad // NS, hid), jnp.float32)
    dsti4 = grid2mesh_edge_indices_dst.reshape(NW, NPH, n_chunks, CHUNK)

    e_ps = []
    for p in range(NPH):
        sp = _make_gather(n_edge, epw_total, p, n_chunks)(
            psrc, pdst, grid2mesh_edge_indices_src,
            grid2mesh_edge_indices_dst)

        e_spec = pl.BlockSpec((pe, hid), lambda i, p=p: (i * NPH + p, 0))
        e_p = pl.pallas_call(
            _edge_body,
            grid=(NW,),
            in_specs=[e_spec, _rows(pe), _full((hid, hid)),
                      _full((1, hid)), _full((hid, hid)), _full((1, hid)),
                      _full((1, hid)), _full((1, hid))],
            out_specs=_rows(pe),
            out_shape=jax.ShapeDtypeStruct((n_ph_edges, hid), jnp.float32),
        )(grid2mesh_edge_features, sp, We1_e, r(be1), We2, r(be2),
          r(ge), r(bbe))

        e_ps.append(e_p)

    sgroups = [(0, 1, 2), (3, 4)]
    partials = [
        _make_scatter(n_mesh_pad, g, n_chunks)(
            *[e_ps[p] for p in g], dsti4, zeros_tile)
        for g in sgroups
    ]

    br_n = 2000
    part_specs = []
    for _ in range(len(partials)):
        for c in range(NC):
            part_specs.append(
                pl.BlockSpec((1, br_n, hid), lambda i, c=c: (c, i, 0)))
    mesh_out = pl.pallas_call(
        _node_body,
        grid=(n_mesh // br_n,),
        in_specs=part_specs + [_rows(br_n), _full((hid, hid)),
                               _full((hid, hid)), _full((1, hid)),
                               _full((hid, hid)), _full((1, hid)),
                               _full((1, hid)), _full((1, hid))],
        out_specs=_rows(br_n),
        out_shape=jax.ShapeDtypeStruct((n_mesh, hid), jnp.float32),
    )(*[pt for pt in partials for _ in range(NC)],
      mesh_node_features, Wn1_a, Wn1_m, r(bn1), Wn2, r(bn2), r(gn), r(bbn))

    return (grid_out, mesh_out)
